# baseline (speedup 1.0000x reference)
import functools

import jax
import jax.numpy as jnp
from jax.experimental import pallas as pl
from jax.experimental.pallas import tpu as pltpu

_BATCH_PER_STEP = 2


def _fused_kernel(x1_ref, x2_ref,
                  w1_ref, b1_ref, w2_ref, b2_ref,
                  wc1_ref, bc1_ref, wc2_ref, bc2_ref, ones_ref,
                  o1_ref, o2_ref, xb_sc, *, n_j, lambda_c, lambda_s):
    C = x1_ref.shape[1]
    bf16 = jnp.bfloat16

    for j in range(n_j):
        x1 = x1_ref[j]
        x2 = x2_ref[j]

        xb_sc[0:C] = x1.astype(bf16)
        xb_sc[C:2 * C] = x2.astype(bf16)

        sums = jnp.dot(xb_sc[...], ones_ref[...],
                       preferred_element_type=jnp.float32)
        max1 = jnp.max(x1, axis=1, keepdims=True)
        max2 = jnp.max(x2, axis=1, keepdims=True)
        y = jnp.concatenate([sums, max1, max2], axis=0)
        h = (jnp.dot(w1_ref[...], y.astype(bf16),
                     preferred_element_type=jnp.float32)
             + b1_ref[...])
        h = jnp.maximum(h, 0.0)
        z = jax.nn.sigmoid(
            jnp.dot(w2_ref[...], h.astype(bf16),
                    preferred_element_type=jnp.float32)
            + b2_ref[...])
        cw0 = lambda_c * z[0:C]
        cw1 = lambda_c * z[C:2 * C]

        hs = (jnp.dot(wc1_ref[...], xb_sc[...],
                      preferred_element_type=jnp.float32)
              + bc1_ref[...])
        hs = jnp.maximum(hs, 0.0).astype(bf16)
        s = jax.nn.sigmoid(
            jnp.dot(wc2_ref[...], hs, preferred_element_type=jnp.float32)
            + bc2_ref[...])
        s0 = lambda_s * s[0:1]
        s1 = lambda_s * s[1:2]

        o1_ref[j] = x1 + (cw1 + s1) * x2
        o2_ref[j] = x2 + (cw0 + s0) * x1


def kernel(x1, x2, w1, b1, w2, b2, wc1, bc1, wc2, bc2):
    B, C, H, W = x1.shape
    HW = H * W
    lambda_c = 0.5
    lambda_s = 0.5
    bf16 = jnp.bfloat16
    x1r = x1.reshape(B, C, HW)
    x2r = x2.reshape(B, C, HW)

    n_j = _BATCH_PER_STEP if B % _BATCH_PER_STEP == 0 else 1
    n_steps = B // n_j

    hid_c = w1.shape[1]
    scale = jnp.concatenate([jnp.full((2 * C, 1), 1.0 / HW, jnp.float32),
                             jnp.ones((2 * C, 1), jnp.float32)], axis=0)
    w1t = (w1 * scale).T.astype(bf16)
    b1c = b1.reshape(hid_c, 1)

    w2t = w2.T.astype(bf16)
    b2c = b2.reshape(2 * C, 1)

    hid_s = wc1.shape[1]
    wc1t = wc1.T.astype(bf16)
    bc1c = bc1.reshape(hid_s, 1)

    wc2t = wc2.T.astype(bf16)
    bc2c = bc2.reshape(2, 1)

    ones_col = jnp.ones((HW, 1), bf16)

    img_spec = pl.BlockSpec((n_j, C, HW), lambda b: (b, 0, 0))

    def const2d(shape):
        return pl.BlockSpec(shape, lambda b: (0, 0))

    o1, o2 = pl.pallas_call(
        functools.partial(_fused_kernel, n_j=n_j,
                          lambda_c=lambda_c, lambda_s=lambda_s),
        out_shape=(jax.ShapeDtypeStruct((B, C, HW), x1.dtype),
                   jax.ShapeDtypeStruct((B, C, HW), x1.dtype)),
        grid=(n_steps,),
        in_specs=[
            img_spec, img_spec,
            const2d((hid_c, 4 * C)), const2d((hid_c, 1)),
            const2d((2 * C, hid_c)), const2d((2 * C, 1)),
            const2d((hid_s, 2 * C)), const2d((hid_s, 1)),
            const2d((2, hid_s)), const2d((2, 1)),
            const2d((HW, 1)),
        ],
        out_specs=[img_spec, img_spec],
        scratch_shapes=[pltpu.VMEM((2 * C, HW), jnp.bfloat16)],
        compiler_params=pltpu.CompilerParams(
            dimension_semantics=("parallel",)),
    )(x1r, x2r, w1t, b1c, w2t, b2c, wc1t, bc1c, wc2t, bc2c, ones_col)

    return o1.reshape(B, C, H, W), o2.reshape(B, C, H, W)


# speedup vs baseline: 1.0372x; 1.0372x over previous
import functools

import jax
import jax.numpy as jnp
from jax.experimental import pallas as pl
from jax.experimental.pallas import tpu as pltpu

_BATCH_PER_STEP = 4


def _fused_kernel(x1_ref, x2_ref,
                  w1_ref, b1_ref, w2_ref, b2_ref,
                  wc1_ref, bc1_ref, wc2_ref, bc2_ref,
                  o1_ref, o2_ref, xb_sc, *, n_j, lambda_c, lambda_s):
    C = x1_ref.shape[1]
    bf16 = jnp.bfloat16

    for j in range(n_j):
        x1 = x1_ref[j]
        x2 = x2_ref[j]

        sum1 = jnp.sum(x1, axis=1, keepdims=True)
        sum2 = jnp.sum(x2, axis=1, keepdims=True)
        max1 = jnp.max(x1, axis=1, keepdims=True)
        max2 = jnp.max(x2, axis=1, keepdims=True)
        y = jnp.concatenate([sum1, sum2, max1, max2], axis=0)
        h = (jnp.dot(w1_ref[...], y.astype(bf16),
                     preferred_element_type=jnp.float32)
             + b1_ref[...])
        h = jnp.maximum(h, 0.0)
        z = jax.nn.sigmoid(
            jnp.dot(w2_ref[...], h.astype(bf16),
                    preferred_element_type=jnp.float32)
            + b2_ref[...])
        cw0 = lambda_c * z[0:C]
        cw1 = lambda_c * z[C:2 * C]

        xb_sc[0:C] = x1.astype(bf16)
        xb_sc[C:2 * C] = x2.astype(bf16)
        hs = (jnp.dot(wc1_ref[...], xb_sc[...],
                      preferred_element_type=jnp.float32)
              + bc1_ref[...])
        hs = jnp.maximum(hs, 0.0).astype(bf16)
        s = jax.nn.sigmoid(
            jnp.dot(wc2_ref[...], hs, preferred_element_type=jnp.float32)
            + bc2_ref[...])
        s0 = lambda_s * s[0:1]
        s1 = lambda_s * s[1:2]

        o1_ref[j] = x1 + (cw1 + s1) * x2
        o2_ref[j] = x2 + (cw0 + s0) * x1


def kernel(x1, x2, w1, b1, w2, b2, wc1, bc1, wc2, bc2):
    B, C, H, W = x1.shape
    HW = H * W
    lambda_c = 0.5
    lambda_s = 0.5
    bf16 = jnp.bfloat16
    x1r = x1.reshape(B, C, HW)
    x2r = x2.reshape(B, C, HW)

    n_j = _BATCH_PER_STEP if B % _BATCH_PER_STEP == 0 else 1
    n_steps = B // n_j

    hid_c = w1.shape[1]
    scale = jnp.concatenate([jnp.full((2 * C, 1), 1.0 / HW, jnp.float32),
                             jnp.ones((2 * C, 1), jnp.float32)], axis=0)
    w1t = (w1 * scale).T.astype(bf16)
    b1c = b1.reshape(hid_c, 1)

    w2t = w2.T.astype(bf16)
    b2c = b2.reshape(2 * C, 1)

    hid_s = wc1.shape[1]
    wc1t = wc1.T.astype(bf16)
    bc1c = bc1.reshape(hid_s, 1)

    wc2t = wc2.T.astype(bf16)
    bc2c = bc2.reshape(2, 1)

    img_spec = pl.BlockSpec((n_j, C, HW), lambda b: (b, 0, 0))

    def const2d(shape):
        return pl.BlockSpec(shape, lambda b: (0, 0))

    o1, o2 = pl.pallas_call(
        functools.partial(_fused_kernel, n_j=n_j,
                          lambda_c=lambda_c, lambda_s=lambda_s),
        out_shape=(jax.ShapeDtypeStruct((B, C, HW), x1.dtype),
                   jax.ShapeDtypeStruct((B, C, HW), x1.dtype)),
        grid=(n_steps,),
        in_specs=[
            img_spec, img_spec,
            const2d((hid_c, 4 * C)), const2d((hid_c, 1)),
            const2d((2 * C, hid_c)), const2d((2 * C, 1)),
            const2d((hid_s, 2 * C)), const2d((hid_s, 1)),
            const2d((2, hid_s)), const2d((2, 1)),
        ],
        out_specs=[img_spec, img_spec],
        scratch_shapes=[pltpu.VMEM((2 * C, HW), jnp.bfloat16)],
        compiler_params=pltpu.CompilerParams(
            dimension_semantics=("parallel",)),
    )(x1r, x2r, w1t, b1c, w2t, b2c, wc1t, bc1c, wc2t, bc2c)

    return o1.reshape(B, C, H, W), o2.reshape(B, C, H, W)
